# baseline (speedup 1.0000x reference)
import jax
import jax.numpy as jnp
from jax import lax
from jax.experimental import pallas as pl
from jax.experimental.pallas import tpu as pltpu
from jax.experimental.pallas import tpu_sc as plsc

X, Y, Z, B, C = 200, 200, 8, 2, 64
P = 540672

NC, NS = 2, 16
W = 5
RPC = (X // W) // NC
CPR = Y * B * Z
NR = W * CPR
DUMMY = NR
ACC_ROWS = NR + 8
CHUNK = 128
CHUNKS_PER_TILE = P // CHUNK // NS
ZROWS = 125
YC = 25
YH = Y // 2


def _body(feats, gx, gy, gz, gb, out,
          acc, gxb, gyb, gzb, gbb, idxb, fb, zbuf, rbuf, obuf):
    c = lax.axis_index("c")
    s = lax.axis_index("s")

    def zb_body(i, _):
        for k in range(C // 16):
            zbuf[i, pl.ds(k * 16, 16)] = jnp.zeros((16,), jnp.float32)
        return 0
    lax.fori_loop(0, ZROWS, zb_body, 0)

    def region_body(r, _):
        x0 = c * (X // NC) + r * W

        for k in range(NR // NS // ZROWS):
            pltpu.sync_copy(zbuf,
                            acc.at[pl.ds(s * (NR // NS) + k * ZROWS, ZROWS)])
        plsc.subcore_barrier()

        def chunk_body(t, _):
            p0 = s * (CHUNKS_PER_TILE * CHUNK) + t * CHUNK
            pltpu.sync_copy(gx.at[pl.ds(p0, CHUNK)], gxb)
            pltpu.sync_copy(gy.at[pl.ds(p0, CHUNK)], gyb)
            pltpu.sync_copy(gz.at[pl.ds(p0, CHUNK)], gzb)
            pltpu.sync_copy(gb.at[pl.ds(p0, CHUNK)], gbb)
            pltpu.sync_copy(feats.at[pl.ds(p0, CHUNK)], fb)
            for i in range(CHUNK // 16):
                sl = pl.ds(i * 16, 16)
                xv = gxb[sl]
                lr = ((xv - x0) * CPR + gyb[sl] * (B * Z)
                      + gbb[sl] * Z + gzb[sl])
                inr = (xv >= x0) & (xv < x0 + W)
                idxb[sl] = jnp.where(inr, lr, DUMMY)
            pltpu.sync_copy(fb, acc.at[idxb], add=True)
            return 0
        lax.fori_loop(0, CHUNKS_PER_TILE, chunk_body, 0)
        plsc.subcore_barrier()

        def do_item(m):
            xx = m // B
            bb = m % B
            xo = (X - 1) - (x0 + xx)
            for yc in range(Y // YC):
                y0 = yc * YC
                pltpu.sync_copy(
                    acc.at[pl.ds(xx * CPR + y0 * (B * Z), YC * B * Z)],
                    rbuf)

                def cell_body(j, _):
                    yo_l = (Y - 1) - y0 - j
                    yos = jnp.full((16,), yo_l, jnp.int32)
                    base = j * (B * Z) + bb * Z
                    for c16 in range(C // 16):
                        cs = pl.ds(c16 * 16, 16)
                        v = rbuf[base, cs]
                        for zz in range(1, Z):
                            v = jnp.maximum(v, rbuf[base + zz, cs])
                        cidx = c16 * 16 + lax.iota(jnp.int32, 16)
                        plsc.store_scatter(obuf, [cidx, yos], v)
                    return 0
                lax.fori_loop(0, YC, cell_body, 0)
            pltpu.sync_copy(obuf, out.at[bb, :, xo, :])

        @pl.when(s < W * B)
        def _item():
            do_item(s)

        plsc.subcore_barrier()
        return 0

    lax.fori_loop(0, RPC, region_body, 0)


def kernel(feats, gx, gy, gz, gb):
    mesh = plsc.VectorSubcoreMesh(core_axis_name="c", subcore_axis_name="s")
    run = pl.kernel(
        _body,
        out_type=jax.ShapeDtypeStruct((B, C, X, Y), jnp.float32),
        mesh=mesh,
        scratch_types=[
            pltpu.VMEM_SHARED((ACC_ROWS, C), jnp.float32),
            pltpu.VMEM((CHUNK,), jnp.int32),
            pltpu.VMEM((CHUNK,), jnp.int32),
            pltpu.VMEM((CHUNK,), jnp.int32),
            pltpu.VMEM((CHUNK,), jnp.int32),
            pltpu.VMEM((CHUNK,), jnp.int32),
            pltpu.VMEM((CHUNK, C), jnp.float32),
            pltpu.VMEM((ZROWS, C), jnp.float32),
            pltpu.VMEM((YC * B * Z, C), jnp.float32),
            pltpu.VMEM((C, Y), jnp.float32),
        ],
        compiler_params=pltpu.CompilerParams(use_tc_tiling_on_sc=False,
                                             needs_layout_passes=False),
        name="bev_lift_scatter",
    )
    comb = run(feats, gx, gy, gz, gb)
    return comb.reshape(1, B * C, X, Y)


# speedup vs baseline: 9.7458x; 9.7458x over previous
import jax
import jax.numpy as jnp
from jax import lax
from jax.experimental import pallas as pl
from jax.experimental.pallas import tpu as pltpu
from jax.experimental.pallas import tpu_sc as plsc

X, Y, Z, B, C = 200, 200, 8, 2, 64
P = 540672

NC, NS = 2, 16
W = 5
NREG = X // W
RPC = NREG // NC
CPR = Y * B * Z
NR = W * CPR
DUMMY = NR
ACC_ROWS = NR + 8
RSH = 14
LRMASK = (1 << RSH) - 1
PPT = P // NS
FIRE = 64
STG = 256
GPB = 4
NB = PPT // (16 * GPB)
YC = 25
RBROWS = YC * Z


def _pack_body(gx, gy, gz, gb, out):
    xv = gx[...]
    reg = xv // W
    lrow = (xv % W) * CPR + gb[...] * (Y * Z) + gy[...] * Z + gz[...]
    out[...] = (reg << RSH) | lrow


def _pack_ranks(gx, gy, gz, gb):
    rows = P // 128
    blk = rows // 8
    grid_spec = pl.GridSpec(
        grid=(8,),
        in_specs=[pl.BlockSpec((blk, 128), lambda i: (i, 0))] * 4,
        out_specs=pl.BlockSpec((blk, 128), lambda i: (i, 0)),
    )
    f = pl.pallas_call(
        _pack_body,
        grid_spec=grid_spec,
        out_shape=jax.ShapeDtypeStruct((rows, 128), jnp.int32),
    )
    r2 = lambda a: a.reshape(rows, 128)
    return f(r2(gx), r2(gy), r2(gz), r2(gb)).reshape(P)


def _body(feats, packed, out,
          acc, rkall, rvstg, pidstg, lrfire, pidfire, fb,
          rbuf, obuf, sem0, gsem):
    c = lax.axis_index("c")
    s = lax.axis_index("s")
    iota = lax.iota(jnp.int32, 16)
    pbase = s * PPT

    pltpu.async_copy(packed.at[pl.ds(pbase, PPT)], rkall, sem0).wait()

    def complete_fire():
        pltpu.make_async_copy(feats.at[pidfire], fb, gsem).wait()
        pltpu.sync_copy(fb, acc.at[lrfire], add=True)

    def fire_batch(pending):
        @pl.when(pending == 1)
        def _():
            complete_fire()
        for k in range(FIRE // 16):
            sl = pl.ds(k * 16, 16)
            lrfire[sl] = rvstg[sl] & LRMASK
            pidfire[sl] = pidstg[sl]
        pltpu.async_copy(feats.at[pidfire], fb, gsem)

    def region_body(r, _):
        target = c * RPC + r
        x0 = target * W

        def zb_body(i, _):
            for k in range(C // 16):
                rbuf[i, pl.ds(k * 16, 16)] = jnp.zeros((16,), jnp.float32)
            return 0
        lax.fori_loop(0, RBROWS, zb_body, 0)

        for k in range(NR // NS // RBROWS):
            pltpu.sync_copy(rbuf,
                            acc.at[pl.ds(s * (NR // NS) + k * RBROWS,
                                         RBROWS)])
        plsc.subcore_barrier()

        def batch_body(bb_, carry):
            cur, pending = carry
            for g in range(GPB):
                sl = pl.ds(bb_ * (16 * GPB) + g * 16, 16)
                rv = rkall[sl]
                inr = (rv >> RSH) == target
                pid = pbase + bb_ * (16 * GPB) + g * 16 + iota
                plsc.store_compressed(rvstg.at[pl.ds(cur, 16)], rv,
                                      mask=inr)
                plsc.store_compressed(pidstg.at[pl.ds(cur, 16)], pid,
                                      mask=inr)
                pcnt = plsc.all_reduce_population_count(inr)
                cur = cur + pcnt[0]

            @pl.when(cur >= FIRE)
            def _fire():
                fire_batch(pending)
                for k in range(FIRE // 16):
                    dsl = pl.ds(k * 16, 16)
                    ssl = pl.ds(FIRE + k * 16, 16)
                    rvstg[dsl] = rvstg[ssl]
                    pidstg[dsl] = pidstg[ssl]
            fired = cur >= FIRE
            return (jnp.where(fired, cur - FIRE, cur),
                    jnp.where(fired, 1, pending))

        cur, pending = lax.fori_loop(0, NB, batch_body, (0, 0))

        dummyv = jnp.full((16,), DUMMY, jnp.int32)
        zerov = jnp.zeros((16,), jnp.int32)
        for k in range(FIRE // 16):
            rvstg[pl.ds(cur + k * 16, 16)] = dummyv
            pidstg[pl.ds(cur + k * 16, 16)] = zerov
        fire_batch(pending)
        complete_fire()
        plsc.subcore_barrier()

        def do_item(m):
            xx = m // B
            bbv = m % B
            xo = (X - 1) - (x0 + xx)
            base = xx * CPR + bbv * (Y * Z)
            for yc in range(Y // YC):
                y0 = yc * YC
                pltpu.sync_copy(acc.at[pl.ds(base + y0 * Z, RBROWS)], rbuf)

                def cell_body(j, _):
                    yo_l = (Y - 1) - y0 - j
                    yos = jnp.full((16,), yo_l, jnp.int32)
                    rbase = j * Z
                    for c16 in range(C // 16):
                        cs = pl.ds(c16 * 16, 16)
                        v = rbuf[rbase, cs]
                        for zz in range(1, Z):
                            v = jnp.maximum(v, rbuf[rbase + zz, cs])
                        cidx = c16 * 16 + iota
                        plsc.store_scatter(obuf, [cidx, yos], v)
                    return 0
                lax.fori_loop(0, YC, cell_body, 0)
            pltpu.sync_copy(obuf, out.at[bbv, :, xo, :])

        @pl.when(s < W * B)
        def _item():
            do_item(s)

        plsc.subcore_barrier()
        return 0

    lax.fori_loop(0, RPC, region_body, 0)


def kernel(feats, gx, gy, gz, gb):
    packed = _pack_ranks(gx, gy, gz, gb)
    mesh = plsc.VectorSubcoreMesh(core_axis_name="c", subcore_axis_name="s")
    run = pl.kernel(
        _body,
        out_type=jax.ShapeDtypeStruct((B, C, X, Y), jnp.float32),
        mesh=mesh,
        scratch_types=[
            pltpu.VMEM_SHARED((ACC_ROWS, C), jnp.float32),
            pltpu.VMEM((PPT,), jnp.int32),
            pltpu.VMEM((STG,), jnp.int32),
            pltpu.VMEM((STG,), jnp.int32),
            pltpu.VMEM((FIRE,), jnp.int32),
            pltpu.VMEM((FIRE,), jnp.int32),
            pltpu.VMEM((FIRE, C), jnp.float32),
            pltpu.VMEM((RBROWS, C), jnp.float32),
            pltpu.VMEM((C, Y), jnp.float32),
            pltpu.SemaphoreType.DMA,
            pltpu.SemaphoreType.DMA,
        ],
        compiler_params=pltpu.CompilerParams(use_tc_tiling_on_sc=False,
                                             needs_layout_passes=False),
        name="bev_lift_scatter",
    )
    comb = run(feats, packed)
    return comb.reshape(1, B * C, X, Y)


# speedup vs baseline: 10.6186x; 1.0896x over previous
import jax
import jax.numpy as jnp
from jax import lax
from jax.experimental import pallas as pl
from jax.experimental.pallas import tpu as pltpu
from jax.experimental.pallas import tpu_sc as plsc

X, Y, Z, B, C = 200, 200, 8, 2, 64
P = 540672

NC, NS = 2, 16
W = 5
NREG = X // W
RPC = NREG // NC
CPR = Y * B * Z
NR = W * CPR
DUMMY = NR
ACC_ROWS = NR + 8
RSH = 14
LRMASK = (1 << RSH) - 1
PPT = P // NS
FIRE = 64
STG = 256
GPB = 4
NB = PPT // (16 * GPB)
YC = 20
RBROWS = YC * Z


def _pack_body(gx, gy, gz, gb, out):
    xv = gx[...]
    reg = xv // W
    lrow = (xv % W) * CPR + gb[...] * (Y * Z) + gy[...] * Z + gz[...]
    out[...] = (reg << RSH) | lrow


def _pack_ranks(gx, gy, gz, gb):
    rows = P // 128
    blk = rows // 8
    grid_spec = pl.GridSpec(
        grid=(8,),
        in_specs=[pl.BlockSpec((blk, 128), lambda i: (i, 0))] * 4,
        out_specs=pl.BlockSpec((blk, 128), lambda i: (i, 0)),
    )
    f = pl.pallas_call(
        _pack_body,
        grid_spec=grid_spec,
        out_shape=jax.ShapeDtypeStruct((rows, 128), jnp.int32),
    )
    r2 = lambda a: a.reshape(rows, 128)
    return f(r2(gx), r2(gy), r2(gz), r2(gb)).reshape(P)


def _body(feats, packed, out,
          acc, rkall, rvstg, pidstg, lrf0, pidf0, lrf1, pidf1, fb0, fb1,
          rbuf, obuf, sem0, gsem0, gsem1):
    c = lax.axis_index("c")
    s = lax.axis_index("s")
    iota = lax.iota(jnp.int32, 16)
    pbase = s * PPT

    pltpu.async_copy(packed.at[pl.ds(pbase, PPT)], rkall, sem0).wait()

    def complete_fire(lrf, pidf, fbx, gsx):
        pltpu.make_async_copy(feats.at[pidf], fbx, gsx).wait()
        pltpu.sync_copy(fbx, acc.at[lrf], add=True)

    def stage_and_issue(lrf, pidf, fbx, gsx):
        for k in range(FIRE // 16):
            sl = pl.ds(k * 16, 16)
            lrf[sl] = rvstg[sl] & LRMASK
            pidf[sl] = pidstg[sl]
        pltpu.async_copy(feats.at[pidf], fbx, gsx)

    def fire_batch(pending, p):
        @pl.when(jnp.logical_and(pending == 2, p == 0))
        def _():
            complete_fire(lrf0, pidf0, fb0, gsem0)

        @pl.when(jnp.logical_and(pending == 2, p == 1))
        def _():
            complete_fire(lrf1, pidf1, fb1, gsem1)

        @pl.when(p == 0)
        def _():
            stage_and_issue(lrf0, pidf0, fb0, gsem0)

        @pl.when(p == 1)
        def _():
            stage_and_issue(lrf1, pidf1, fb1, gsem1)

    def region_body(r, _):
        target = c * RPC + r
        x0 = target * W

        def zb_body(i, _):
            for k in range(C // 16):
                rbuf[i, pl.ds(k * 16, 16)] = jnp.zeros((16,), jnp.float32)
            return 0
        lax.fori_loop(0, RBROWS, zb_body, 0)

        for k in range(NR // NS // RBROWS):
            pltpu.sync_copy(rbuf,
                            acc.at[pl.ds(s * (NR // NS) + k * RBROWS,
                                         RBROWS)])
        rem = (NR // NS) % RBROWS
        if rem:
            pltpu.sync_copy(
                rbuf.at[pl.ds(0, rem)],
                acc.at[pl.ds(s * (NR // NS)
                             + (NR // NS // RBROWS) * RBROWS, rem)])
        plsc.subcore_barrier()

        def batch_body(bb_, carry):
            cur, pending, p = carry
            for g in range(GPB):
                sl = pl.ds(bb_ * (16 * GPB) + g * 16, 16)
                rv = rkall[sl]
                inr = (rv >> RSH) == target
                pid = pbase + bb_ * (16 * GPB) + g * 16 + iota
                plsc.store_compressed(rvstg.at[pl.ds(cur, 16)], rv,
                                      mask=inr)
                plsc.store_compressed(pidstg.at[pl.ds(cur, 16)], pid,
                                      mask=inr)
                pcnt = plsc.all_reduce_population_count(inr)
                cur = cur + pcnt[0]

            @pl.when(cur >= FIRE)
            def _fire():
                fire_batch(pending, p)
                for k in range(FIRE // 16):
                    dsl = pl.ds(k * 16, 16)
                    ssl = pl.ds(FIRE + k * 16, 16)
                    rvstg[dsl] = rvstg[ssl]
                    pidstg[dsl] = pidstg[ssl]
            fired = cur >= FIRE
            return (jnp.where(fired, cur - FIRE, cur),
                    jnp.where(fired, jnp.minimum(pending + 1, 2), pending),
                    jnp.where(fired, 1 - p, p))

        cur, pending, p = lax.fori_loop(0, NB, batch_body, (0, 0, 0))

        dummyv = jnp.full((16,), DUMMY, jnp.int32)
        zerov = jnp.zeros((16,), jnp.int32)
        for k in range(FIRE // 16):
            rvstg[pl.ds(cur + k * 16, 16)] = dummyv
            pidstg[pl.ds(cur + k * 16, 16)] = zerov
        fire_batch(pending, p)
        pend2 = jnp.minimum(pending + 1, 2)

        @pl.when(jnp.logical_and(pend2 == 2, p == 0))
        def _():
            complete_fire(lrf1, pidf1, fb1, gsem1)

        @pl.when(jnp.logical_and(pend2 == 2, p == 1))
        def _():
            complete_fire(lrf0, pidf0, fb0, gsem0)

        @pl.when(p == 0)
        def _():
            complete_fire(lrf0, pidf0, fb0, gsem0)

        @pl.when(p == 1)
        def _():
            complete_fire(lrf1, pidf1, fb1, gsem1)
        plsc.subcore_barrier()

        def do_item(m):
            xx = m // B
            bbv = m % B
            xo = (X - 1) - (x0 + xx)
            base = xx * CPR + bbv * (Y * Z)
            for yc in range(Y // YC):
                y0 = yc * YC
                pltpu.sync_copy(acc.at[pl.ds(base + y0 * Z, RBROWS)], rbuf)

                def cell_body(j, _):
                    yo_l = (Y - 1) - y0 - j
                    yos = jnp.full((16,), yo_l, jnp.int32)
                    rbase = j * Z
                    for c16 in range(C // 16):
                        cs = pl.ds(c16 * 16, 16)
                        v = rbuf[rbase, cs]
                        for zz in range(1, Z):
                            v = jnp.maximum(v, rbuf[rbase + zz, cs])
                        cidx = c16 * 16 + iota
                        plsc.store_scatter(obuf, [cidx, yos], v)
                    return 0
                lax.fori_loop(0, YC, cell_body, 0)
            pltpu.sync_copy(obuf, out.at[bbv, :, xo, :])

        @pl.when(s < W * B)
        def _item():
            do_item(s)

        plsc.subcore_barrier()
        return 0

    lax.fori_loop(0, RPC, region_body, 0)


def kernel(feats, gx, gy, gz, gb):
    packed = _pack_ranks(gx, gy, gz, gb)
    mesh = plsc.VectorSubcoreMesh(core_axis_name="c", subcore_axis_name="s")
    run = pl.kernel(
        _body,
        out_type=jax.ShapeDtypeStruct((B, C, X, Y), jnp.float32),
        mesh=mesh,
        scratch_types=[
            pltpu.VMEM_SHARED((ACC_ROWS, C), jnp.float32),
            pltpu.VMEM((PPT,), jnp.int32),
            pltpu.VMEM((STG,), jnp.int32),
            pltpu.VMEM((STG,), jnp.int32),
            pltpu.VMEM((FIRE,), jnp.int32),
            pltpu.VMEM((FIRE,), jnp.int32),
            pltpu.VMEM((FIRE,), jnp.int32),
            pltpu.VMEM((FIRE,), jnp.int32),
            pltpu.VMEM((FIRE, C), jnp.float32),
            pltpu.VMEM((FIRE, C), jnp.float32),
            pltpu.VMEM((RBROWS, C), jnp.float32),
            pltpu.VMEM((C, Y), jnp.float32),
            pltpu.SemaphoreType.DMA,
            pltpu.SemaphoreType.DMA,
            pltpu.SemaphoreType.DMA,
        ],
        compiler_params=pltpu.CompilerParams(use_tc_tiling_on_sc=False,
                                             needs_layout_passes=False),
        name="bev_lift_scatter",
    )
    comb = run(feats, packed)
    return comb.reshape(1, B * C, X, Y)
